# baseline (speedup 1.0000x reference)
import jax
import jax.numpy as jnp
from jax.experimental import pallas as pl


def _proj_kernel(x_ref, wt_ref, y_ref):
    y_ref[...] = jnp.dot(x_ref[...], wt_ref[...],
                         preferred_element_type=jnp.float32)


def _main_kernel(a_ref, y_ref, b_ref, o_ref):
    acc = jnp.dot(a_ref[...], y_ref[...],
                  preferred_element_type=jnp.float32)
    o_ref[...] = jnp.maximum(acc + b_ref[...], 0.0)


def kernel(node_features, adjacency_matrix, W, b):
    n, d_in = node_features.shape
    d_out = W.shape[0]

    y = pl.pallas_call(
        _proj_kernel,
        out_shape=jax.ShapeDtypeStruct((n, d_out), jnp.float32),
    )(node_features, W.T)

    bm = 400
    out = pl.pallas_call(
        _main_kernel,
        grid=(n // bm,),
        in_specs=[
            pl.BlockSpec((bm, n), lambda i: (i, 0)),
            pl.BlockSpec((n, d_out), lambda i: (0, 0)),
            pl.BlockSpec((1, d_out), lambda i: (0, 0)),
        ],
        out_specs=pl.BlockSpec((bm, d_out), lambda i: (i, 0)),
        out_shape=jax.ShapeDtypeStruct((n, d_out), jnp.float32),
    )(adjacency_matrix, y, b.reshape(1, d_out))
    return out


# speedup vs baseline: 1.0095x; 1.0095x over previous
import jax
import jax.numpy as jnp
from jax.experimental import pallas as pl


def _proj_kernel(x_ref, wt_ref, y_ref):
    y_ref[...] = jnp.dot(x_ref[...], wt_ref[...],
                         preferred_element_type=jnp.float32).astype(jnp.bfloat16)


def _main_kernel(a_ref, y_ref, b_ref, o_ref):
    acc = jnp.dot(a_ref[...].astype(jnp.bfloat16), y_ref[...],
                  preferred_element_type=jnp.float32)
    o_ref[...] = jnp.maximum(acc + b_ref[...], 0.0)


def kernel(node_features, adjacency_matrix, W, b):
    n, d_in = node_features.shape
    d_out = W.shape[0]

    y = pl.pallas_call(
        _proj_kernel,
        out_shape=jax.ShapeDtypeStruct((n, d_out), jnp.bfloat16),
    )(node_features, W.T)

    bm = 400
    out = pl.pallas_call(
        _main_kernel,
        grid=(n // bm,),
        in_specs=[
            pl.BlockSpec((bm, n), lambda i: (i, 0)),
            pl.BlockSpec((n, d_out), lambda i: (0, 0)),
            pl.BlockSpec((1, d_out), lambda i: (0, 0)),
        ],
        out_specs=pl.BlockSpec((bm, d_out), lambda i: (i, 0)),
        out_shape=jax.ShapeDtypeStruct((n, d_out), jnp.float32),
    )(adjacency_matrix, y, b.reshape(1, d_out))
    return out
